# baseline (speedup 1.0000x reference)
import jax
import jax.numpy as jnp
import numpy as np
from jax.experimental import pallas as pl
from jax.experimental.pallas import tpu as pltpu

_ALPHA = 4


def _make_body(idx):
    def _body(in_hbm, fast_hbm, slow_hbm, sem_fast, sem_slow):
        fast_dma = pltpu.make_async_copy(in_hbm, fast_hbm, sem_fast)
        fast_dma.start()
        slow_dmas = []
        for j, t in enumerate(idx):
            d = pltpu.make_async_copy(
                in_hbm.at[:, pl.ds(int(t), 1)],
                slow_hbm.at[:, pl.ds(j, 1)],
                sem_slow,
            )
            d.start()
            slow_dmas.append(d)
        for d in slow_dmas:
            d.wait()
        fast_dma.wait()

    return _body


def kernel(frames):
    C, T, H, W = frames.shape
    n = T // _ALPHA
    idx = np.linspace(0.0, T - 1, n).astype(np.int32)

    fast, slow = pl.pallas_call(
        _make_body(idx),
        in_specs=[pl.BlockSpec(memory_space=pltpu.MemorySpace.HBM)],
        out_specs=[
            pl.BlockSpec(memory_space=pltpu.MemorySpace.HBM),
            pl.BlockSpec(memory_space=pltpu.MemorySpace.HBM),
        ],
        out_shape=[
            jax.ShapeDtypeStruct((C, T, H, W), frames.dtype),
            jax.ShapeDtypeStruct((C, n, H, W), frames.dtype),
        ],
        scratch_shapes=[pltpu.SemaphoreType.DMA, pltpu.SemaphoreType.DMA],
    )(frames)
    return (slow, fast)


# speedup vs baseline: 51.6225x; 51.6225x over previous
import jax
import jax.numpy as jnp
import numpy as np
from jax.experimental import pallas as pl
from jax.experimental.pallas import tpu as pltpu

_ALPHA = 4
_NBUF = 4


def _make_body(idx, n):
    offs = [int(t) - _ALPHA * j for j, t in enumerate(idx)]

    def _body(in_hbm, fast_hbm, slow_hbm, bufs, sem_in, sem_fast, sem_slow):
        def in_dma(j):
            return pltpu.make_async_copy(
                in_hbm.at[:, pl.ds(j * _ALPHA, _ALPHA)],
                bufs.at[j % _NBUF],
                sem_in.at[j % _NBUF],
            )

        def fast_dma(j):
            return pltpu.make_async_copy(
                bufs.at[j % _NBUF],
                fast_hbm.at[:, pl.ds(j * _ALPHA, _ALPHA)],
                sem_fast.at[j % _NBUF],
            )

        def slow_dma(j):
            return pltpu.make_async_copy(
                bufs.at[j % _NBUF, :, pl.ds(offs[j], 1)],
                slow_hbm.at[:, pl.ds(j, 1)],
                sem_slow.at[j % _NBUF],
            )

        for j in range(min(_NBUF, n)):
            in_dma(j).start()
        for j in range(n):
            if j >= 1 and j - 1 + _NBUF < n:
                fast_dma(j - 1).wait()
                slow_dma(j - 1).wait()
                in_dma(j - 1 + _NBUF).start()
            in_dma(j).wait()
            fast_dma(j).start()
            slow_dma(j).start()
        for j in range(max(0, n - _NBUF), n):
            fast_dma(j).wait()
            slow_dma(j).wait()

    return _body


def kernel(frames):
    C, T, H, W = frames.shape
    n = T // _ALPHA
    idx = np.linspace(0.0, T - 1, n).astype(np.int32)
    assert all(_ALPHA * j <= int(t) < _ALPHA * (j + 1) for j, t in enumerate(idx))

    fast, slow = pl.pallas_call(
        _make_body(idx, n),
        in_specs=[pl.BlockSpec(memory_space=pltpu.MemorySpace.HBM)],
        out_specs=[
            pl.BlockSpec(memory_space=pltpu.MemorySpace.HBM),
            pl.BlockSpec(memory_space=pltpu.MemorySpace.HBM),
        ],
        out_shape=[
            jax.ShapeDtypeStruct((C, T, H, W), frames.dtype),
            jax.ShapeDtypeStruct((C, n, H, W), frames.dtype),
        ],
        scratch_shapes=[
            pltpu.VMEM((_NBUF, C, _ALPHA, H, W), frames.dtype),
            pltpu.SemaphoreType.DMA((_NBUF,)),
            pltpu.SemaphoreType.DMA((_NBUF,)),
            pltpu.SemaphoreType.DMA((_NBUF,)),
        ],
    )(frames)
    return (slow, fast)
